# baseline (speedup 1.0000x reference)
import jax
import jax.numpy as jnp
from jax.experimental import pallas as pl
from jax.experimental.pallas import tpu as pltpu

N = 4096
D = 512
BJ = 512
NJ = N // BJ
NP = 2
BI = N // NP
NSPLIT = 8
NH = N // NSPLIT
NEG_SLOPE = 0.2
LOG2E = 1.4426950408889634
NEG_BIG = -1e30


def _lrelu(x):
    return jnp.maximum(x, NEG_SLOPE * x)


def _body(x_ref, w_ref, asrc_ref, adst_ref, *rest):
    a_refs = rest[:NSPLIT]
    bias_ref, out_ref, xp_ref, as_ref, ad_ref, gmax_ref = rest[NSPLIT:]
    t = pl.program_id(0)

    @pl.when(t < NP)
    def _proj():
        i = t
        xp = jax.lax.dot_general(
            x_ref[...].astype(jnp.bfloat16), w_ref[...].astype(jnp.bfloat16),
            (((1,), (0,)), ((), ())), preferred_element_type=jnp.float32)
        xp_ref[pl.ds(i * BI, BI), :] = xp
        as_blk = jax.lax.dot_general(
            xp, asrc_ref[...] * LOG2E, (((1,), (0,)), ((), ())),
            preferred_element_type=jnp.float32)
        as_ref[pl.ds(i * BI, BI), :] = as_blk
        ad_ref[0:1, pl.ds(i * BI, BI)] = jax.lax.dot_general(
            adst_ref[...] * LOG2E, xp, (((0,), (1,)), ((), ())),
            preferred_element_type=jnp.float32)
        prev = jnp.where(i == 0, jnp.full((1, 1), NEG_BIG, jnp.float32),
                         gmax_ref[...])
        gmax_ref[...] = jnp.maximum(prev, jnp.max(as_blk))

    @pl.when(t >= NP)
    def _agg():
        j = t - NP
        ad_row = ad_ref[0:1, pl.ds(j * BJ, BJ)]
        m2 = _lrelu(gmax_ref[...] + ad_row)
        ones = jnp.ones((NH, 1), jnp.float32)

        def _contrib(a_h, k):
            z = as_ref[pl.ds(k * NH, NH), :] + ad_row
            e2 = _lrelu(z)
            p = a_h[...] * jnp.exp2(e2 - m2)
            o = jax.lax.dot_general(
                p, xp_ref[pl.ds(k * NH, NH), :], (((0,), (0,)), ((), ())),
                preferred_element_type=jnp.float32)
            sc = jax.lax.dot_general(
                p, ones, (((0,), (0,)), ((), ())),
                preferred_element_type=jnp.float32)
            return o, sc

        parts = [_contrib(r, k) for k, r in enumerate(a_refs)]
        out = parts[0][0]
        s = parts[0][1]
        for o, sc in parts[1:]:
            out = out + o
            s = s + sc
        s_safe = jnp.where(s > 0.0, s, 1.0)
        out_ref[...] = jnp.maximum(out / s_safe + bias_ref[...], 0.0)


@jax.jit
def kernel(A, X, W, a_src, a_dst, bias):
    d_in = X.shape[1]
    a_specs = [
        pl.BlockSpec((NH, BJ), lambda t, k=k: (k, jnp.maximum(t - NP, 0)))
        for k in range(NSPLIT)
    ]
    out = pl.pallas_call(
        _body,
        grid=(NP + NJ,),
        in_specs=[
            pl.BlockSpec((BI, d_in), lambda t: (jnp.minimum(t, NP - 1), 0)),
            pl.BlockSpec((d_in, D), lambda t: (0, 0)),
            pl.BlockSpec((D, 1), lambda t: (0, 0)),
            pl.BlockSpec((D, 1), lambda t: (0, 0)),
        ] + a_specs + [
            pl.BlockSpec((1, D), lambda t: (0, 0)),
        ],
        out_specs=pl.BlockSpec((BJ, D), lambda t: (jnp.maximum(t - NP, 0), 0)),
        out_shape=jax.ShapeDtypeStruct((N, D), jnp.float32),
        scratch_shapes=[
            pltpu.VMEM((N, D), jnp.float32),
            pltpu.VMEM((N, 1), jnp.float32),
            pltpu.VMEM((1, N), jnp.float32),
            pltpu.VMEM((1, 1), jnp.float32),
        ],
        compiler_params=pltpu.CompilerParams(
            dimension_semantics=("arbitrary",)),
    )(X, W, a_src.reshape(D, 1), a_dst.reshape(D, 1),
      *([A] * NSPLIT), bias.reshape(1, D))

    return out


# speedup vs baseline: 1.0105x; 1.0105x over previous
import jax
import jax.numpy as jnp
from jax.experimental import pallas as pl
from jax.experimental.pallas import tpu as pltpu

N = 4096
D = 512
BJ = 512
NJ = N // BJ
NP = 4
BI = N // NP
NSPLIT = 8
NH = N // NSPLIT
NEG_SLOPE = 0.2
LOG2E = 1.4426950408889634
NEG_BIG = -1e30


def _lrelu(x):
    return jnp.maximum(x, NEG_SLOPE * x)


def _body(x_ref, w_ref, asrc_ref, adst_ref, *rest):
    a_refs = rest[:NSPLIT]
    bias_ref, out_ref, xp_ref, as_ref, ad_ref, gmax_ref = rest[NSPLIT:]
    t = pl.program_id(0)

    @pl.when(t < NP)
    def _proj():
        i = t
        xp = jax.lax.dot_general(
            x_ref[...].astype(jnp.bfloat16), w_ref[...].astype(jnp.bfloat16),
            (((1,), (0,)), ((), ())), preferred_element_type=jnp.float32)
        xp_ref[pl.ds(i * BI, BI), :] = xp
        as_blk = jax.lax.dot_general(
            xp, asrc_ref[...] * LOG2E, (((1,), (0,)), ((), ())),
            preferred_element_type=jnp.float32)
        as_ref[pl.ds(i * BI, BI), :] = as_blk
        ad_ref[0:1, pl.ds(i * BI, BI)] = jax.lax.dot_general(
            adst_ref[...] * LOG2E, xp, (((0,), (1,)), ((), ())),
            preferred_element_type=jnp.float32)
        prev = jnp.where(i == 0, jnp.full((1, 1), NEG_BIG, jnp.float32),
                         gmax_ref[...])
        gmax_ref[...] = jnp.maximum(prev, jnp.max(as_blk))

    @pl.when(t >= NP)
    def _agg():
        j = t - NP
        ad_row = ad_ref[0:1, pl.ds(j * BJ, BJ)]
        m2 = _lrelu(gmax_ref[...] + ad_row)
        ones = jnp.ones((NH, 1), jnp.float32)

        def _contrib(a_h, k):
            z = as_ref[pl.ds(k * NH, NH), :] + ad_row
            e2 = _lrelu(z)
            p = a_h[...] * jnp.exp2(e2 - m2)
            o = jax.lax.dot_general(
                p, xp_ref[pl.ds(k * NH, NH), :], (((0,), (0,)), ((), ())),
                preferred_element_type=jnp.float32)
            sc = jax.lax.dot_general(
                p, ones, (((0,), (0,)), ((), ())),
                preferred_element_type=jnp.float32)
            return o, sc

        parts = [_contrib(r, k) for k, r in enumerate(a_refs)]
        out = parts[0][0]
        s = parts[0][1]
        for o, sc in parts[1:]:
            out = out + o
            s = s + sc
        s_safe = jnp.where(s > 0.0, s, 1.0)
        out_ref[...] = jnp.maximum(out / s_safe + bias_ref[...], 0.0)


@jax.jit
def kernel(A, X, W, a_src, a_dst, bias):
    d_in = X.shape[1]
    a_specs = [
        pl.BlockSpec((NH, BJ), lambda t, k=k: (k, jnp.maximum(t - NP, 0)))
        for k in range(NSPLIT)
    ]
    out = pl.pallas_call(
        _body,
        grid=(NP + NJ,),
        in_specs=[
            pl.BlockSpec((BI, d_in), lambda t: (jnp.minimum(t, NP - 1), 0)),
            pl.BlockSpec((d_in, D), lambda t: (0, 0)),
            pl.BlockSpec((D, 1), lambda t: (0, 0)),
            pl.BlockSpec((D, 1), lambda t: (0, 0)),
        ] + a_specs + [
            pl.BlockSpec((1, D), lambda t: (0, 0)),
        ],
        out_specs=pl.BlockSpec((BJ, D), lambda t: (jnp.maximum(t - NP, 0), 0)),
        out_shape=jax.ShapeDtypeStruct((N, D), jnp.float32),
        scratch_shapes=[
            pltpu.VMEM((N, D), jnp.float32),
            pltpu.VMEM((N, 1), jnp.float32),
            pltpu.VMEM((1, N), jnp.float32),
            pltpu.VMEM((1, 1), jnp.float32),
        ],
        compiler_params=pltpu.CompilerParams(
            dimension_semantics=("arbitrary",)),
    )(X, W, a_src.reshape(D, 1), a_dst.reshape(D, 1),
      *([A] * NSPLIT), bias.reshape(1, D))

    return out
